# baseline (speedup 1.0000x reference)
import jax
import jax.numpy as jnp
from jax import lax
from jax.experimental import pallas as pl
from jax.experimental.pallas import tpu as pltpu
from jax.experimental.pallas import tpu_sc as plsc

N_NODES = 10000
N_EDGES = 320000
D = 128

NC = 2
NS = 16
NW = NC * NS

CH = 128
CPW = 80
GC = 40
E_PAD = NW * CPW * CH
NPAD = E_PAD - N_EDGES
N_TRASH = 128
Y_ROWS = N_NODES + N_TRASH

SEG = 624
TAIL = N_NODES - NS * SEG


def _mm_body(x_ref, w_ref, o_ref):
    o_ref[...] = lax.dot_general(
        x_ref[...], w_ref[...], (((1,), (1,)), ((), ())),
        preferred_element_type=jnp.float32)


def _matmul(x, W):
    return pl.pallas_call(
        _mm_body,
        grid=(10,),
        in_specs=[
            pl.BlockSpec((1000, D), lambda i: (i, 0)),
            pl.BlockSpec((D, D), lambda i: (0, 0)),
        ],
        out_specs=pl.BlockSpec((1000, D), lambda i: (i, 0)),
        out_shape=jax.ShapeDtypeStruct((N_NODES, D), jnp.float32),
    )(x, W)


def _agg_body(h_hbm, src_hbm, dst_hbm, out_hbm,
              src_v, dst_v, rows0, rows1, gsem0, gsem1, y_sh):
    c = lax.axis_index("c")
    s = lax.axis_index("s")
    wid = c * NS + s
    epw = CPW * CH

    zeros16 = jnp.zeros((16,), jnp.float32)

    def zrow(i, _):
        for j in range(D // 16):
            rows0[i, pl.ds(j * 16, 16)] = zeros16
        return 0

    lax.fori_loop(0, CH, zrow, 0)

    base_row = s * SEG
    for k in range(SEG // CH):
        pltpu.sync_copy(rows0, y_sh.at[pl.ds(base_row + k * CH, CH)])
    rem = SEG % CH
    pltpu.sync_copy(rows0.at[pl.ds(0, rem)],
                    y_sh.at[pl.ds(base_row + (SEG // CH) * CH, rem)])

    @pl.when(s == NS - 1)
    def _():
        pltpu.sync_copy(rows0.at[pl.ds(0, TAIL + 8)],
                        y_sh.at[pl.ds(NS * SEG, TAIL + 8)])

    plsc.subcore_barrier()

    bufs = ((rows0, gsem0), (rows1, gsem1))

    for grp in range(CPW // GC):
        gbase = wid * epw + grp * GC * CH
        pltpu.sync_copy(src_hbm.at[pl.ds(gbase, GC * CH)], src_v)
        pltpu.sync_copy(dst_hbm.at[pl.ds(gbase, GC * CH)], dst_v)

        pltpu.async_copy(h_hbm.at[src_v.at[pl.ds(0, CH)]], rows0, gsem0)
        pltpu.async_copy(h_hbm.at[src_v.at[pl.ds(CH, CH)]], rows1, gsem1)

        def body(g2, _):
            for b in range(2):
                rows, gsem = bufs[b]
                lc = g2 * 2 + b
                pltpu.make_async_copy(
                    h_hbm.at[src_v.at[pl.ds(lc * CH, CH)]], rows, gsem).wait()
                pltpu.sync_copy(rows, y_sh.at[dst_v.at[pl.ds(lc * CH, CH)]],
                                add=True)

                @pl.when(lc < GC - 2)
                def _():
                    pltpu.async_copy(
                        h_hbm.at[src_v.at[pl.ds((lc + 2) * CH, CH)]],
                        rows, gsem)

            return 0

        lax.fori_loop(0, GC // 2, body, 0)

    plsc.subcore_barrier()

    pltpu.sync_copy(y_sh.at[pl.ds(base_row, SEG)],
                    out_hbm.at[c, pl.ds(base_row, SEG)])

    @pl.when(s == NS - 1)
    def _():
        pltpu.sync_copy(y_sh.at[pl.ds(NS * SEG, TAIL)],
                        out_hbm.at[c, pl.ds(NS * SEG, TAIL)])


def _aggregate(h, src, dst):
    mesh = plsc.VectorSubcoreMesh(
        core_axis_name="c", subcore_axis_name="s", num_cores=NC,
        num_subcores=NS)
    f = pl.kernel(
        _agg_body,
        out_type=jax.ShapeDtypeStruct((NC, N_NODES, D), jnp.float32),
        mesh=mesh,
        scratch_types=[
            pltpu.VMEM((GC * CH,), jnp.int32),
            pltpu.VMEM((GC * CH,), jnp.int32),
            pltpu.VMEM((CH, D), jnp.float32),
            pltpu.VMEM((CH, D), jnp.float32),
            pltpu.SemaphoreType.DMA,
            pltpu.SemaphoreType.DMA,
            pltpu.VMEM_SHARED((Y_ROWS, D), jnp.float32),
        ],
    )
    return f(h, src, dst)


def _add_body(a_ref, o_ref):
    o_ref[...] = a_ref[0] + a_ref[1]


def _combine(reps):
    return pl.pallas_call(
        _add_body,
        grid=(10,),
        in_specs=[pl.BlockSpec((NC, 1000, D), lambda i: (0, i, 0))],
        out_specs=pl.BlockSpec((1000, D), lambda i: (i, 0)),
        out_shape=jax.ShapeDtypeStruct((N_NODES, D), jnp.float32),
    )(reps)


def kernel(x, edge_index, W):
    h = _matmul(x, W)
    src = jnp.concatenate([edge_index[0], jnp.zeros((NPAD,), jnp.int32)])
    trash = N_NODES + jnp.arange(NPAD, dtype=jnp.int32) % N_TRASH
    dst = jnp.concatenate([edge_index[1], trash])
    reps = _aggregate(h, src, dst)
    return _combine(reps)


# speedup vs baseline: 3.4008x; 3.4008x over previous
import jax
import jax.numpy as jnp
from jax import lax
from jax.experimental import pallas as pl
from jax.experimental.pallas import tpu as pltpu
from jax.experimental.pallas import tpu_sc as plsc

N_NODES = 10000
N_EDGES = 320000
D = 128

NC = 2
NS = 16
NW = NC * NS

CH = 128
CPW = 80
GC = 40
E_PAD = NW * CPW * CH
NPAD = E_PAD - N_EDGES
N_TRASH = 128
Y_ROWS = N_NODES + N_TRASH

SEG = 624
TAIL = N_NODES - NS * SEG


def _mm_body(x_ref, w_ref, o_ref):
    o_ref[...] = lax.dot_general(
        x_ref[...], w_ref[...], (((1,), (1,)), ((), ())),
        preferred_element_type=jnp.float32)


def _matmul(x, W):
    return pl.pallas_call(
        _mm_body,
        grid=(10,),
        in_specs=[
            pl.BlockSpec((1000, D), lambda i: (i, 0)),
            pl.BlockSpec((D, D), lambda i: (0, 0)),
        ],
        out_specs=pl.BlockSpec((1000, D), lambda i: (i, 0)),
        out_shape=jax.ShapeDtypeStruct((N_NODES, D), jnp.float32),
    )(x, W)


def _agg_body(h_hbm, src_hbm, dst_hbm, out_hbm,
              src_v, dst_v, rows0, rows1, gsem0, gsem1, y_sh):
    c = lax.axis_index("c")
    s = lax.axis_index("s")
    wid = c * NS + s
    epw = CPW * CH

    zeros16 = jnp.zeros((16,), jnp.float32)

    def zrow(i, _):
        for j in range(D // 16):
            rows0[i, pl.ds(j * 16, 16)] = zeros16
        return 0

    lax.fori_loop(0, CH, zrow, 0)

    base_row = s * SEG
    for k in range(SEG // CH):
        pltpu.sync_copy(rows0, y_sh.at[pl.ds(base_row + k * CH, CH)])
    rem = SEG % CH
    pltpu.sync_copy(rows0.at[pl.ds(0, rem)],
                    y_sh.at[pl.ds(base_row + (SEG // CH) * CH, rem)])

    @pl.when(s == NS - 1)
    def _():
        pltpu.sync_copy(rows0.at[pl.ds(0, TAIL + 8)],
                        y_sh.at[pl.ds(NS * SEG, TAIL + 8)])

    plsc.subcore_barrier()

    bufs = ((rows0, gsem0), (rows1, gsem1))

    for grp in range(CPW // GC):
        gbase = wid * epw + grp * GC * CH
        pltpu.sync_copy(src_hbm.at[pl.ds(gbase, GC * CH)], src_v)
        pltpu.sync_copy(dst_hbm.at[pl.ds(gbase, GC * CH)], dst_v)

        pltpu.async_copy(h_hbm.at[src_v.at[pl.ds(0, CH)]], rows0, gsem0)
        pltpu.async_copy(h_hbm.at[src_v.at[pl.ds(CH, CH)]], rows1, gsem1)

        def body(g2, _):
            for b in range(2):
                rows, gsem = bufs[b]
                lc = g2 * 2 + b
                pltpu.make_async_copy(
                    h_hbm.at[src_v.at[pl.ds(lc * CH, CH)]], rows, gsem).wait()
                pltpu.sync_copy(rows, y_sh.at[dst_v.at[pl.ds(lc * CH, CH)]],
                                add=True)

                @pl.when(lc < GC - 2)
                def _():
                    pltpu.async_copy(
                        h_hbm.at[src_v.at[pl.ds((lc + 2) * CH, CH)]],
                        rows, gsem)

            return 0

        lax.fori_loop(0, GC // 2, body, 0)

    plsc.subcore_barrier()

    pltpu.sync_copy(y_sh.at[pl.ds(base_row, SEG)],
                    out_hbm.at[c, pl.ds(base_row, SEG)])

    @pl.when(s == NS - 1)
    def _():
        pltpu.sync_copy(y_sh.at[pl.ds(NS * SEG, TAIL)],
                        out_hbm.at[c, pl.ds(NS * SEG, TAIL)])


def _aggregate(h, src, dst):
    mesh = plsc.VectorSubcoreMesh(
        core_axis_name="c", subcore_axis_name="s", num_cores=NC,
        num_subcores=NS)
    f = pl.kernel(
        _agg_body,
        out_type=jax.ShapeDtypeStruct((NC, N_NODES, D), jnp.float32),
        mesh=mesh,
        scratch_types=[
            pltpu.VMEM((GC * CH,), jnp.int32),
            pltpu.VMEM((GC * CH,), jnp.int32),
            pltpu.VMEM((CH, D), jnp.float32),
            pltpu.VMEM((CH, D), jnp.float32),
            pltpu.SemaphoreType.DMA,
            pltpu.SemaphoreType.DMA,
            pltpu.VMEM_SHARED((Y_ROWS, D), jnp.float32),
        ],
    )
    return f(h, src, dst)


def _add_body(a_ref, o_ref):
    o_ref[...] = a_ref[0] + a_ref[1]


def _combine(reps):
    return pl.pallas_call(
        _add_body,
        grid=(10,),
        in_specs=[pl.BlockSpec((NC, 1000, D), lambda i: (0, i, 0))],
        out_specs=pl.BlockSpec((1000, D), lambda i: (i, 0)),
        out_shape=jax.ShapeDtypeStruct((N_NODES, D), jnp.float32),
    )(reps)


def kernel(x, edge_index, W):
    h = _matmul(x, W)
    pad_src = jnp.arange(NPAD, dtype=jnp.int32) % N_NODES
    src = jnp.concatenate([edge_index[0], pad_src])
    trash = N_NODES + jnp.arange(NPAD, dtype=jnp.int32) % N_TRASH
    dst = jnp.concatenate([edge_index[1], trash])
    reps = _aggregate(h, src, dst)
    return _combine(reps)


# speedup vs baseline: 3.4089x; 1.0024x over previous
import jax
import jax.numpy as jnp
from jax import lax
from jax.experimental import pallas as pl
from jax.experimental.pallas import tpu as pltpu
from jax.experimental.pallas import tpu_sc as plsc

N_NODES = 10000
N_EDGES = 320000
D = 128

NC = 2
NS = 16
NW = NC * NS

CH = 128
EPW = N_EDGES // NW
GROUPS = ((0, 40, 40 * CH), (40 * CH, 38, 38 * CH + 16))
TAIL_OFF = 40 * CH + 38 * CH
TAIL_E = EPW - TAIL_OFF

SEG = 624
TAIL_R = N_NODES - NS * SEG


def _mm_body(x_ref, w_ref, o_ref):
    o_ref[...] = lax.dot_general(
        x_ref[...], w_ref[...], (((1,), (1,)), ((), ())),
        preferred_element_type=jnp.float32)


def _matmul(x, W):
    return pl.pallas_call(
        _mm_body,
        grid=(10,),
        in_specs=[
            pl.BlockSpec((1000, D), lambda i: (i, 0)),
            pl.BlockSpec((D, D), lambda i: (0, 0)),
        ],
        out_specs=pl.BlockSpec((1000, D), lambda i: (i, 0)),
        out_shape=jax.ShapeDtypeStruct((N_NODES, D), jnp.float32),
    )(x, W)


def _agg_body(h_hbm, src_hbm, dst_hbm, out_hbm,
              src_v, dst_v, rows0, rows1, gsem0, gsem1, y_sh):
    c = lax.axis_index("c")
    s = lax.axis_index("s")
    wid = c * NS + s
    ebase = wid * EPW

    zeros16 = jnp.zeros((16,), jnp.float32)

    def zrow(i, _):
        for j in range(D // 16):
            rows0[i, pl.ds(j * 16, 16)] = zeros16
        return 0

    lax.fori_loop(0, CH, zrow, 0)

    base_row = s * SEG
    for k in range(SEG // CH):
        pltpu.sync_copy(rows0, y_sh.at[pl.ds(base_row + k * CH, CH)])
    rem = SEG % CH
    pltpu.sync_copy(rows0.at[pl.ds(0, rem)],
                    y_sh.at[pl.ds(base_row + (SEG // CH) * CH, rem)])

    @pl.when(s == NS - 1)
    def _():
        pltpu.sync_copy(rows0.at[pl.ds(0, TAIL_R)],
                        y_sh.at[pl.ds(NS * SEG, TAIL_R)])

    plsc.subcore_barrier()

    bufs = ((rows0, gsem0), (rows1, gsem1))

    for goff, nfull, stage in GROUPS:
        pltpu.sync_copy(src_hbm.at[pl.ds(ebase + goff, stage)],
                        src_v.at[pl.ds(0, stage)])
        pltpu.sync_copy(dst_hbm.at[pl.ds(ebase + goff, stage)],
                        dst_v.at[pl.ds(0, stage)])

        pltpu.async_copy(h_hbm.at[src_v.at[pl.ds(0, CH)]], rows0, gsem0)
        pltpu.async_copy(h_hbm.at[src_v.at[pl.ds(CH, CH)]], rows1, gsem1)

        def body(g2, _):
            for b in range(2):
                rows, gsem = bufs[b]
                lc = g2 * 2 + b
                pltpu.make_async_copy(
                    h_hbm.at[src_v.at[pl.ds(lc * CH, CH)]], rows, gsem).wait()
                pltpu.sync_copy(rows, y_sh.at[dst_v.at[pl.ds(lc * CH, CH)]],
                                add=True)

                @pl.when(lc < nfull - 2)
                def _():
                    pltpu.async_copy(
                        h_hbm.at[src_v.at[pl.ds((lc + 2) * CH, CH)]],
                        rows, gsem)

            return 0

        lax.fori_loop(0, nfull // 2, body, 0)

    toff = TAIL_OFF - GROUPS[1][0]
    pltpu.sync_copy(h_hbm.at[src_v.at[pl.ds(toff, TAIL_E)]],
                    rows0.at[pl.ds(0, TAIL_E)])
    pltpu.sync_copy(rows0.at[pl.ds(0, TAIL_E)],
                    y_sh.at[dst_v.at[pl.ds(toff, TAIL_E)]], add=True)

    plsc.subcore_barrier()

    pltpu.sync_copy(y_sh.at[pl.ds(base_row, SEG)],
                    out_hbm.at[c, pl.ds(base_row, SEG)])

    @pl.when(s == NS - 1)
    def _():
        pltpu.sync_copy(y_sh.at[pl.ds(NS * SEG, TAIL_R)],
                        out_hbm.at[c, pl.ds(NS * SEG, TAIL_R)])


def _aggregate(h, src, dst):
    mesh = plsc.VectorSubcoreMesh(
        core_axis_name="c", subcore_axis_name="s", num_cores=NC,
        num_subcores=NS)
    f = pl.kernel(
        _agg_body,
        out_type=jax.ShapeDtypeStruct((NC, N_NODES, D), jnp.float32),
        mesh=mesh,
        scratch_types=[
            pltpu.VMEM((40 * CH,), jnp.int32),
            pltpu.VMEM((40 * CH,), jnp.int32),
            pltpu.VMEM((CH, D), jnp.float32),
            pltpu.VMEM((CH, D), jnp.float32),
            pltpu.SemaphoreType.DMA,
            pltpu.SemaphoreType.DMA,
            pltpu.VMEM_SHARED((N_NODES, D), jnp.float32),
        ],
    )
    return f(h, src, dst)


def _add_body(a_ref, o_ref):
    o_ref[...] = a_ref[0] + a_ref[1]


def _combine(reps):
    return pl.pallas_call(
        _add_body,
        grid=(10,),
        in_specs=[pl.BlockSpec((NC, 1000, D), lambda i: (0, i, 0))],
        out_specs=pl.BlockSpec((1000, D), lambda i: (i, 0)),
        out_shape=jax.ShapeDtypeStruct((N_NODES, D), jnp.float32),
    )(reps)


def kernel(x, edge_index, W):
    h = _matmul(x, W)
    reps = _aggregate(h, edge_index[0], edge_index[1])
    return _combine(reps)
